# baseline (device time: 7581 ns/iter reference)
import jax
import jax.numpy as jnp
from jax import lax
from jax.experimental import pallas as pl
from jax.experimental.pallas import tpu as pltpu


def kernel(u):
    n0, n1, n2 = u.shape
    dtype = u.dtype

    def body(u_ref, out_ref, sx, sy, sz, rx, ry, rz, send_sems, recv_sems):
        my_x = lax.axis_index("x")
        my_y = lax.axis_index("y")
        my_z = lax.axis_index("z")

        barrier_sem = pltpu.get_barrier_semaphore()
        for dev in [
            (1 - my_x, my_y, my_z),
            (my_x, 1 - my_y, my_z),
            (my_x, my_y, 1 - my_z),
        ]:
            pl.semaphore_signal(
                barrier_sem, inc=1,
                device_id=dev, device_id_type=pl.DeviceIdType.MESH,
            )

        sx[...] = jnp.where(my_x == 0, u_ref[n0 - 1, :, :], u_ref[0, :, :])
        sy[...] = jnp.where(my_y == 0, u_ref[:, n1 - 1, :], u_ref[:, 0, :])
        sz[...] = jnp.where(my_z == 0, u_ref[:, :, n2 - 1], u_ref[:, :, 0])

        pl.semaphore_wait(barrier_sem, 3)

        rdmas = []
        for a, (sbuf, rbuf, dev) in enumerate([
            (sx, rx, (1 - my_x, my_y, my_z)),
            (sy, ry, (my_x, 1 - my_y, my_z)),
            (sz, rz, (my_x, my_y, 1 - my_z)),
        ]):
            rdma = pltpu.make_async_remote_copy(
                src_ref=sbuf,
                dst_ref=rbuf,
                send_sem=send_sems.at[a],
                recv_sem=recv_sems.at[a],
                device_id=dev,
                device_id_type=pl.DeviceIdType.MESH,
            )
            rdma.start()
            rdmas.append(rdma)

        uv = u_ref[...]
        z0 = jnp.zeros((1, n1, n2), dtype)
        z1 = jnp.zeros((n0, 1, n2), dtype)
        z2 = jnp.zeros((n0, n1, 1), dtype)
        v = (
            jnp.concatenate([z0, uv[:-1]], axis=0)
            + jnp.concatenate([uv[1:], z0], axis=0)
            + jnp.concatenate([z1, uv[:, :-1, :]], axis=1)
            + jnp.concatenate([uv[:, 1:, :], z1], axis=1)
            + jnp.concatenate([z2, uv[:, :, :-1]], axis=2)
            + jnp.concatenate([uv[:, :, 1:], z2], axis=2)
            - 6.0 * uv
        )

        out_ref[...] = v

        for rdma in rdmas:
            rdma.wait_recv()

        @pl.when(my_x == 0)
        def _():
            out_ref[n0 - 1, :, :] = out_ref[n0 - 1, :, :] + rx[...]

        @pl.when(my_x == 1)
        def _():
            out_ref[0, :, :] = out_ref[0, :, :] + rx[...]

        @pl.when(my_y == 0)
        def _():
            out_ref[:, n1 - 1, :] = out_ref[:, n1 - 1, :] + ry[...]

        @pl.when(my_y == 1)
        def _():
            out_ref[:, 0, :] = out_ref[:, 0, :] + ry[...]

        @pl.when(my_z == 0)
        def _():
            out_ref[:, :, n2 - 1] = out_ref[:, :, n2 - 1] + rz[...]

        @pl.when(my_z == 1)
        def _():
            out_ref[:, :, 0] = out_ref[:, :, 0] + rz[...]

        @pl.when(my_x == 0)
        def _():
            out_ref[0, :, :] = jnp.zeros((n1, n2), dtype)

        @pl.when(my_x == 1)
        def _():
            out_ref[n0 - 1, :, :] = jnp.zeros((n1, n2), dtype)

        @pl.when(my_y == 0)
        def _():
            out_ref[:, 0, :] = jnp.zeros((n0, n2), dtype)

        @pl.when(my_y == 1)
        def _():
            out_ref[:, n1 - 1, :] = jnp.zeros((n0, n2), dtype)

        @pl.when(my_z == 0)
        def _():
            out_ref[:, :, 0] = jnp.zeros((n0, n1), dtype)

        @pl.when(my_z == 1)
        def _():
            out_ref[:, :, n2 - 1] = jnp.zeros((n0, n1), dtype)

        for rdma in rdmas:
            rdma.wait_send()

    return pl.pallas_call(
        body,
        out_shape=jax.ShapeDtypeStruct((n0, n1, n2), dtype),
        in_specs=[pl.BlockSpec(memory_space=pltpu.VMEM)],
        out_specs=pl.BlockSpec(memory_space=pltpu.VMEM),
        scratch_shapes=[
            pltpu.VMEM((n1, n2), dtype),
            pltpu.VMEM((n0, n2), dtype),
            pltpu.VMEM((n0, n1), dtype),
            pltpu.VMEM((n1, n2), dtype),
            pltpu.VMEM((n0, n2), dtype),
            pltpu.VMEM((n0, n1), dtype),
            pltpu.SemaphoreType.DMA((3,)),
            pltpu.SemaphoreType.DMA((3,)),
        ],
        compiler_params=pltpu.CompilerParams(collective_id=0),
    )(u)
